# baseline (speedup 1.0000x reference)
import functools

import jax
import jax.numpy as jnp
from jax.experimental import pallas as pl

_NUM_BUCKETS = 1024.0
_BLOCK_N = 8000


def _lsh_block_kernel(x_ref, p_ref, b_ref, o_ref):
    h = jax.lax.dot_general(
        x_ref[...], p_ref[...],
        dimension_numbers=(((1,), (1,)), ((), ())),
        preferred_element_type=jnp.float32,
    )
    h = h + b_ref[...]
    o_ref[...] = jnp.floor(h * (1.0 / _NUM_BUCKETS)).astype(jnp.int32)


@jax.jit
def kernel(x, projections, biases):
    n, emb = x.shape
    num_hashes = projections.shape[0]
    grid = (pl.cdiv(n, _BLOCK_N),)
    out = pl.pallas_call(
        _lsh_block_kernel,
        grid=grid,
        in_specs=[
            pl.BlockSpec((_BLOCK_N, emb), lambda i: (i, 0)),
            pl.BlockSpec((num_hashes, emb), lambda i: (0, 0)),
            pl.BlockSpec((1, num_hashes), lambda i: (0, 0)),
        ],
        out_specs=pl.BlockSpec((_BLOCK_N, num_hashes), lambda i: (i, 0)),
        out_shape=jax.ShapeDtypeStruct((n, num_hashes), jnp.int32),
    )(x, projections, biases.reshape(1, num_hashes))
    return out
